# baseline (speedup 1.0000x reference)
import functools

import jax
import jax.numpy as jnp
from jax import lax
from jax.experimental import pallas as pl
from jax.experimental.pallas import tpu as pltpu
from jax.experimental.pallas import tpu_sc as plsc

VOCAB = 100000
D = 1024
B = 128
L = 32
N = B * L
NC, NS, LANES = 2, 16, 16
NW = NC * NS
ROWS_PER_W = N // NW
IDROWS = ROWS_PER_W // L
BUFROWS = 32
VPR = D // LANES
NBUF = 3
SCHED = ((0, 32), (32, 32), (64, 32), (96, 32))
NCHUNK = len(SCHED)
SUB = 8


def _emb_body(ids_hbm, pidx_hbm, emb_hbm, pos_hbm, out_hbm,
              idx_v, pidx_v, pos_v,
              b0, b1, b2,
              g0, g1, g2,
              s0, s1, s2):
    bufs = (b0, b1, b2)
    gsems = (g0, g1, g2)
    ssems = (s0, s1, s2)
    wid = lax.axis_index("s") * NC + lax.axis_index("c")
    base = wid * ROWS_PER_W

    pltpu.sync_copy(ids_hbm.at[pl.ds(wid * IDROWS, IDROWS)], idx_v)
    pltpu.sync_copy(pidx_hbm, pidx_v)
    pos_cp = pltpu.async_copy(pos_hbm.at[pidx_v], pos_v, ssems[0])

    def gather(c):
        off, sz = SCHED[c]
        idx = idx_v.at[off // L, pl.ds(off % L, sz)]
        return pltpu.async_copy(emb_hbm.at[idx], bufs[c % NBUF].at[pl.ds(0, sz)],
                                gsems[c % NBUF])

    gcp = [None] * NBUF
    scp = [None] * NBUF
    for c in range(NBUF - 1):
        gcp[c] = gather(c)
    pos_cp.wait()
    for c in range(NCHUNK):
        off, sz = SCHED[c]
        bi = c % NBUF
        buf = bufs[bi]
        gcp[bi].wait()

        nxt = c + NBUF - 1
        if nxt < NCHUNK:
            nb = nxt % NBUF
            if scp[nb] is not None:
                for cp in scp[nb]:
                    cp.wait()
                scp[nb] = None
            gcp[nb] = gather(nxt)

        subs = []
        for r0 in range(0, sz, SUB):
            @plsc.parallel_loop(0, VPR, unroll=4)
            def col(j):
                sl = pl.ds(j * LANES, LANES)
                pv = pos_v[0, sl]
                for r in range(r0, r0 + SUB):
                    buf[r, sl] = buf[r, sl] + pv

            f = off + r0
            subs.append(pltpu.async_copy(
                buf.at[pl.ds(r0, SUB)],
                out_hbm.at[wid * IDROWS + f // L, pl.ds(f % L, SUB)],
                ssems[bi]))
        scp[bi] = subs
    for cps in scp:
        if cps is not None:
            for cp in cps:
                cp.wait()


_emb_kernel = functools.partial(
    pl.kernel,
    out_type=jax.ShapeDtypeStruct((B, L, D), jnp.float32),
    mesh=plsc.VectorSubcoreMesh(core_axis_name="c", subcore_axis_name="s",
                                num_cores=NC, num_subcores=NS),
    scratch_types=(
        [pltpu.VMEM((IDROWS, L), jnp.int32),
         pltpu.VMEM((1,), jnp.int32),
         pltpu.VMEM((1, D), jnp.float32)]
        + [pltpu.VMEM((BUFROWS, D), jnp.float32)] * NBUF
        + [pltpu.SemaphoreType.DMA] * (2 * NBUF)
    ),
)(_emb_body)


def kernel(gpt_ids, gen_len, emb_table, pos_table):
    pidx = jnp.reshape(jnp.asarray(gen_len, jnp.int32), (1,))
    hidden = _emb_kernel(gpt_ids.astype(jnp.int32), pidx, emb_table, pos_table)
    return hidden, gen_len + 1


# speedup vs baseline: 1.0668x; 1.0668x over previous
import functools

import jax
import jax.numpy as jnp
from jax import lax
from jax.experimental import pallas as pl
from jax.experimental.pallas import tpu as pltpu
from jax.experimental.pallas import tpu_sc as plsc

VOCAB = 100000
D = 1024
B = 128
L = 32
N = B * L
NC, NS, LANES = 2, 16, 16
NW = NC * NS
ROWS_PER_W = N // NW
IDROWS = ROWS_PER_W // L
BUFROWS = 32
VPR = D // LANES
NBUF = 3
SCHED = ((0, 32), (32, 32), (64, 32), (96, 32))
NCHUNK = len(SCHED)
SUB = 8


def _emb_body(ids_hbm, pidx_hbm, emb_hbm, pos_hbm, out_hbm,
              idx_v, pidx_v, pos_v,
              b0, b1, b2,
              g0, g1, g2,
              s0, s1, s2):
    bufs = (b0, b1, b2)
    gsems = (g0, g1, g2)
    ssems = (s0, s1, s2)
    wid = lax.axis_index("s") * NC + lax.axis_index("c")
    base = wid * ROWS_PER_W

    pltpu.sync_copy(ids_hbm.at[pl.ds(wid * IDROWS, IDROWS)], idx_v)
    pltpu.sync_copy(pidx_hbm, pidx_v)
    pos_cp = pltpu.async_copy(pos_hbm.at[pidx_v], pos_v, ssems[0])

    def gather(c):
        off, sz = SCHED[c]
        idx = idx_v.at[off // L, pl.ds(off % L, sz)]
        return pltpu.async_copy(emb_hbm.at[idx], bufs[c % NBUF].at[pl.ds(0, sz)],
                                gsems[c % NBUF])

    gcp = [None] * NBUF
    scp = [None] * NBUF
    for c in range(NBUF - 1):
        gcp[c] = gather(c)
    pos_cp.wait()
    for c in range(NCHUNK):
        off, sz = SCHED[c]
        bi = c % NBUF
        buf = bufs[bi]
        gcp[bi].wait()

        nxt = c + NBUF - 1
        if nxt < NCHUNK:
            nb = nxt % NBUF
            if scp[nb] is not None:
                for cp in scp[nb]:
                    cp.wait()
                scp[nb] = None
            gcp[nb] = gather(nxt)

        subs = []
        for r0 in range(0, sz, SUB):
            @plsc.parallel_loop(0, VPR, unroll=2)
            def col(j):
                sl = pl.ds(j * LANES, LANES)
                pv = pos_v[0, sl]
                for r in range(r0, r0 + SUB):
                    buf[r, sl] = buf[r, sl] + pv

            f = off + r0
            subs.append(pltpu.async_copy(
                buf.at[pl.ds(r0, SUB)],
                out_hbm.at[wid * IDROWS + f // L, pl.ds(f % L, SUB)],
                ssems[bi]))
        scp[bi] = subs
    for cps in scp:
        if cps is not None:
            for cp in cps:
                cp.wait()


_emb_kernel = functools.partial(
    pl.kernel,
    out_type=jax.ShapeDtypeStruct((B, L, D), jnp.float32),
    mesh=plsc.VectorSubcoreMesh(core_axis_name="c", subcore_axis_name="s",
                                num_cores=NC, num_subcores=NS),
    scratch_types=(
        [pltpu.VMEM((IDROWS, L), jnp.int32),
         pltpu.VMEM((1,), jnp.int32),
         pltpu.VMEM((1, D), jnp.float32)]
        + [pltpu.VMEM((BUFROWS, D), jnp.float32)] * NBUF
        + [pltpu.SemaphoreType.DMA] * (2 * NBUF)
    ),
)(_emb_body)


def kernel(gpt_ids, gen_len, emb_table, pos_table):
    pidx = jnp.reshape(jnp.asarray(gen_len, jnp.int32), (1,))
    hidden = _emb_kernel(gpt_ids.astype(jnp.int32), pidx, emb_table, pos_table)
    return hidden, gen_len + 1


# speedup vs baseline: 1.0704x; 1.0033x over previous
import functools

import jax
import jax.numpy as jnp
from jax import lax
from jax.experimental import pallas as pl
from jax.experimental.pallas import tpu as pltpu
from jax.experimental.pallas import tpu_sc as plsc

VOCAB = 100000
D = 1024
B = 128
L = 32
N = B * L
NC, NS, LANES = 2, 16, 16
NW = NC * NS
ROWS_PER_W = N // NW
IDROWS = ROWS_PER_W // L
BUFROWS = 32
VPR = D // LANES
NBUF = 3
SCHED = ((0, 32), (32, 32), (64, 32), (96, 32))
NCHUNK = len(SCHED)
SUB = 8


def _emb_body(ids_hbm, emb_hbm, pos_hbm, out_hbm,
              idx_v, pos_v,
              b0, b1, b2,
              g0, g1, g2,
              s0, s1, s2):
    bufs = (b0, b1, b2)
    gsems = (g0, g1, g2)
    ssems = (s0, s1, s2)
    wid = lax.axis_index("s") * NC + lax.axis_index("c")

    pltpu.sync_copy(ids_hbm.at[wid], idx_v)
    pos_cp = pltpu.async_copy(
        pos_hbm.at[idx_v.at[pl.ds(ROWS_PER_W, 1)]], pos_v, ssems[0])

    def gather(c):
        off, sz = SCHED[c]
        idx = idx_v.at[pl.ds(off, sz)]
        return pltpu.async_copy(emb_hbm.at[idx], bufs[c % NBUF].at[pl.ds(0, sz)],
                                gsems[c % NBUF])

    gcp = [None] * NBUF
    scp = [None] * NBUF
    for c in range(NBUF - 1):
        gcp[c] = gather(c)
    pos_cp.wait()
    for c in range(NCHUNK):
        off, sz = SCHED[c]
        bi = c % NBUF
        buf = bufs[bi]
        gcp[bi].wait()

        nxt = c + NBUF - 1
        if nxt < NCHUNK:
            nb = nxt % NBUF
            if scp[nb] is not None:
                for cp in scp[nb]:
                    cp.wait()
                scp[nb] = None
            gcp[nb] = gather(nxt)

        subs = []
        for r0 in range(0, sz, SUB):
            @plsc.parallel_loop(0, VPR, unroll=2)
            def col(j):
                sl = pl.ds(j * LANES, LANES)
                pv = pos_v[0, sl]
                for r in range(r0, r0 + SUB):
                    buf[r, sl] = buf[r, sl] + pv

            f = off + r0
            subs.append(pltpu.async_copy(
                buf.at[pl.ds(r0, SUB)],
                out_hbm.at[wid * IDROWS + f // L, pl.ds(f % L, SUB)],
                ssems[bi]))
        scp[bi] = subs
    for cps in scp:
        if cps is not None:
            for cp in cps:
                cp.wait()


_emb_kernel = functools.partial(
    pl.kernel,
    out_type=jax.ShapeDtypeStruct((B, L, D), jnp.float32),
    mesh=plsc.VectorSubcoreMesh(core_axis_name="c", subcore_axis_name="s",
                                num_cores=NC, num_subcores=NS),
    scratch_types=(
        [pltpu.VMEM((ROWS_PER_W + 8,), jnp.int32),
         pltpu.VMEM((1, D), jnp.float32)]
        + [pltpu.VMEM((BUFROWS, D), jnp.float32)] * NBUF
        + [pltpu.SemaphoreType.DMA] * (2 * NBUF)
    ),
)(_emb_body)


def kernel(gpt_ids, gen_len, emb_table, pos_table):
    aug = jnp.concatenate(
        [jnp.reshape(gpt_ids, (NW, ROWS_PER_W)).astype(jnp.int32),
         jnp.full((NW, 8), gen_len, jnp.int32)], axis=1)
    hidden = _emb_kernel(aug, emb_table, pos_table)
    return hidden, gen_len + 1
